# baseline (speedup 1.0000x reference)
import functools

import jax
import jax.numpy as jnp
from jax import lax
from jax.experimental import pallas as pl
from jax.experimental.pallas import tpu as pltpu
from jax.experimental.pallas import tpu_sc as plsc

F32 = jnp.float32
I32 = jnp.int32

B = 2
NC = 2
NS = 16
NW = NC * NS


def _mm3(xf, Wn, u, v, wes, rb):
    n, k = xf.shape
    cin = k // B
    c = Wn.shape[1]
    wout = B * c

    def body(x_ref, wn_ref, u_ref, v_ref, ws_ref, o1, o2, o3, o4):
        for b in range(B):
            cs = slice(b * c, (b + 1) * c)
            xb = x_ref[:, b * cin:(b + 1) * cin]
            xw = jnp.dot(xb, wn_ref[...], preferred_element_type=F32)
            xwu = jnp.dot(xw, u_ref[...], preferred_element_type=F32)
            o1[:, cs] = xw
            o2[:, cs] = xwu
            o3[:, cs] = xwu * ws_ref[:, cs]
            o4[:, cs] = jnp.dot(xw, v_ref[...], preferred_element_type=F32)

    outs = [jax.ShapeDtypeStruct((n, wout), F32)] * 4
    return pl.pallas_call(
        body,
        grid=(n // rb,),
        in_specs=[
            pl.BlockSpec((rb, k), lambda i: (i, 0)),
            pl.BlockSpec((cin, c), lambda i: (0, 0)),
            pl.BlockSpec((c, c), lambda i: (0, 0)),
            pl.BlockSpec((c, c), lambda i: (0, 0)),
            pl.BlockSpec((1, wout), lambda i: (0, 0)),
        ],
        out_specs=[pl.BlockSpec((rb, wout), lambda i: (i, 0))] * 4,
        out_shape=outs,
    )(xf, Wn, u, v, wes)


def _gather3(tab1, tab2, tab3, idx, m_pad, ch):
    w = tab1.shape[1]
    rpt = m_pad // NW
    nch = rpt // ch
    mesh = plsc.VectorSubcoreMesh(core_axis_name="c", subcore_axis_name="s")

    @functools.partial(
        pl.kernel,
        mesh=mesh,
        compiler_params=pltpu.CompilerParams(use_tc_tiling_on_sc=False),
        out_type=[jax.ShapeDtypeStruct((m_pad, w), F32)] * 3,
        scratch_types=[
            pltpu.VMEM((ch,), I32),
            pltpu.VMEM((ch, w), F32),
            pltpu.VMEM((ch, w), F32),
            pltpu.VMEM((ch, w), F32),
            pltpu.SemaphoreType.DMA,
            pltpu.SemaphoreType.DMA,
            pltpu.SemaphoreType.DMA,
        ],
    )
    def k(t1, t2, t3, idx_hbm, o1, o2, o3, idx_v, b1, b2, b3, s1, s2, s3):
        cid = lax.axis_index("c")
        sid = lax.axis_index("s")
        base = (cid * NS + sid) * rpt

        def chunk(j, carry):
            off = base + j * ch
            pltpu.sync_copy(idx_hbm.at[pl.ds(off, ch)], idx_v)
            c1 = pltpu.async_copy(t1.at[idx_v], b1, s1)
            c2 = pltpu.async_copy(t2.at[idx_v], b2, s2)
            c3 = pltpu.async_copy(t3.at[idx_v], b3, s3)
            c1.wait()
            c2.wait()
            c3.wait()
            pltpu.sync_copy(b1, o1.at[pl.ds(off, ch)])
            pltpu.sync_copy(b2, o2.at[pl.ds(off, ch)])
            pltpu.sync_copy(b3, o3.at[pl.ds(off, ch)])
            return carry

        lax.fori_loop(0, nch, chunk, 0)

    return k(tab1, tab2, tab3, idx)


def _edge_aggregate(xtu_s, xsv_s, src, tgt, wgt, zrows,
                    nt, ch, n_xtu, n_xsv):
    wc = xtu_s.shape[1]
    wp = wc + 16
    e_pad = src.shape[0]
    tile_e = e_pad // NS
    nch = tile_e // ch
    rpa = zrows.shape[0]
    agr = NS * rpa
    nreg = wc // 16
    mesh = plsc.VectorSubcoreMesh(core_axis_name="c", subcore_axis_name="s")

    assert nch % 2 == 0 and nch >= 4

    @functools.partial(
        pl.kernel,
        mesh=mesh,
        compiler_params=pltpu.CompilerParams(use_tc_tiling_on_sc=False),
        out_type=jax.ShapeDtypeStruct((NC, agr, wp), F32),
        scratch_types=[
            pltpu.VMEM((2, ch), I32),
            pltpu.VMEM((2, ch), I32),
            pltpu.VMEM((2, ch), I32),
            pltpu.VMEM((2, ch), F32),
            pltpu.VMEM((2, ch, wc), F32),
            pltpu.VMEM((2, ch, wc), F32),
            pltpu.VMEM((2, ch, wp), F32),
            pltpu.VMEM_SHARED((agr, wp), F32),
            pltpu.SemaphoreType.DMA,
            pltpu.SemaphoreType.DMA,
            pltpu.SemaphoreType.DMA,
            pltpu.SemaphoreType.DMA,
            pltpu.SemaphoreType.DMA,
            pltpu.SemaphoreType.DMA,
        ],
    )
    def k(xtu, xsv, src_h, tgt_h, w_h, z_h, part,
          tgtn, tgto, srco, wv, bt, bs, msg, acc,
          st0, st1, ss0, ss1, sa0, sa1):
        cid = lax.axis_index("c")
        sid = lax.axis_index("s")
        base = sid * tile_e
        dt = cid * n_xtu
        doff = cid * n_xsv
        st = (st0, st1)
        ss = (ss0, ss1)
        sa = (sa0, sa1)

        pltpu.sync_copy(z_h, acc.at[pl.ds(sid * rpa, rpa)])
        plsc.subcore_barrier()

        one0 = jnp.where(lax.iota(I32, 16) == 0,
                         jnp.float32(1.0), jnp.float32(0.0))

        def init_cnt(e, carry):
            msg[0, e, pl.ds(wc, 16)] = one0
            msg[1, e, pl.ds(wc, 16)] = one0
            return carry

        lax.fori_loop(0, ch, init_cnt, 0)

        def fire(j, p):
            off = base + j * ch
            pltpu.sync_copy(tgt_h.at[pl.ds(off, ch)], tgto.at[p])
            pltpu.sync_copy(src_h.at[pl.ds(off, ch)], srco.at[p])
            pltpu.sync_copy(w_h.at[pl.ds(off, ch)], wv.at[p])
            for i in range(ch // 16):
                sl = pl.ds(16 * i, 16)
                tgto[p, sl] = tgto[p, sl] + dt
                srco[p, sl] = srco[p, sl] + doff
            pltpu.async_copy(xtu.at[tgto.at[p]], bt.at[p], st[p])
            pltpu.async_copy(xsv.at[srco.at[p]], bs.at[p], ss[p])

        def compute(j2, p):
            pltpu.make_async_copy(xtu.at[tgto.at[p]], bt.at[p], st[p]).wait()
            pltpu.make_async_copy(xsv.at[srco.at[p]], bs.at[p], ss[p]).wait()

            @pl.when(j2 >= 1)
            def _drain():
                pltpu.make_async_copy(
                    msg.at[p], acc.at[tgtn.at[p]], sa[p]).wait()

            for i in range(ch // 16):
                sl = pl.ds(16 * i, 16)
                tgtn[p, sl] = tgto[p, sl] - dt

            @plsc.parallel_loop(0, ch // 16, unroll=2 if nreg == 1 else 1)
            def edge_group(g):
                w16 = wv[p, pl.ds(g * 16, 16)]
                for l in range(16):
                    e = g * 16 + l
                    we_s = jnp.take_along_axis(
                        w16, jnp.full((16,), l, dtype=I32), axis=0)
                    for r in range(nreg):
                        xu = bt[p, e, pl.ds(16 * r, 16)]
                        xv = bs[p, e, pl.ds(16 * r, 16)]
                        z = xu * xv * we_s
                        z = jnp.minimum(jnp.maximum(z, -120.0), 120.0)
                        zm = z + 12582912.0
                        f = z - (zm - 12582912.0)
                        ki = lax.bitcast_convert_type(zm, jnp.int32)
                        scale = lax.bitcast_convert_type((ki + 127) << 23, F32)
                        pe = 0.009618129107628477
                        pe = pe * f + 0.05550410866482158
                        pe = pe * f + 0.2402265069591007
                        pe = pe * f + 0.6931471805599453
                        pe = pe * f + 1.0
                        den = 1.0 + pe * scale
                        y = lax.bitcast_convert_type(
                            2129690602
                            - lax.bitcast_convert_type(den, jnp.int32), F32)
                        y = y * (2.0 - den * y)
                        y = y * (2.0 - den * y)
                        msg[p, e, pl.ds(16 * r, 16)] = xv * y

            pltpu.async_copy(msg.at[p], acc.at[tgtn.at[p]], sa[p], add=True)

        fire(0, 0)
        fire(1, 1)

        def pair(j2, carry):
            j = 2 * j2
            compute(j2, 0)

            @pl.when(j + 2 < nch)
            def _f0():
                fire(j + 2, 0)

            compute(j2, 1)

            @pl.when(j + 3 < nch)
            def _f1():
                fire(j + 3, 1)

            return carry

        lax.fori_loop(0, nch // 2, pair, 0)
        pltpu.make_async_copy(msg.at[0], acc.at[tgtn.at[0]], sa0).wait()
        pltpu.make_async_copy(msg.at[1], acc.at[tgtn.at[1]], sa1).wait()
        plsc.subcore_barrier()
        pltpu.sync_copy(acc.at[pl.ds(sid * rpa, rpa)],
                        part.at[cid, pl.ds(sid * rpa, rpa)])

    return k(xtu_s, xsv_s, src, tgt, wgt, zrows)


def _post(xt, xtu, part, nt):
    w = xt.shape[1]

    wc = w // 2

    def body(xt_ref, xtu_ref, p_ref, o_ref):
        sums = jnp.concatenate(
            [p_ref[0][:nt, :wc], p_ref[1][:nt, :wc]], axis=1)
        cnt = p_ref[0][:nt, wc:wc + 1]
        aggr = sums / jnp.maximum(cnt, 1.0)
        o = xtu_ref[:nt] + aggr
        mean = jnp.mean(o, axis=1, keepdims=True)
        var = jnp.mean((o - mean) ** 2, axis=1, keepdims=True)
        h = xt_ref[:nt] + (o - mean) / jnp.sqrt(var + 1e-5)
        o_ref[...] = jnp.where(h >= 0, h, 0.01 * h)

    return pl.pallas_call(
        body,
        out_shape=jax.ShapeDtypeStruct((nt, w), F32),
    )(xt, xtu, part)


def _pad_edges(edge_index, wgt, nt, ch):
    e = edge_index.shape[1]
    tile_e = -(-(e // NS) // (2 * ch)) * (2 * ch)
    pad = NS * tile_e - e
    src = jnp.concatenate([edge_index[0].astype(I32),
                           jnp.zeros((pad,), I32)])
    tgt = jnp.concatenate([edge_index[1].astype(I32),
                           jnp.full((pad,), nt, I32)])
    wp = jnp.concatenate([wgt, jnp.zeros((pad,), F32)])
    return src, tgt, wp


def _pad_idx(idx, m_pad):
    m = idx.shape[0]
    return jnp.concatenate([idx.astype(I32), jnp.zeros((m_pad - m,), I32)])


def _layer(xf, n_nodes, n_tgt, edge_index, edge_weight, res_n_id,
           Wn, We, u, v, rb):
    c = Wn.shape[1]
    w = B * c
    wc = w // 2

    wes = (jnp.tile(We.reshape(-1), B) * (-1.4426950408889634)).reshape(1, w)
    xw, xwu, xwus, xwv = _mm3(xf, Wn, u, v, wes, rb)

    m_pad = -(-n_tgt // (NW * 16)) * (NW * 16)
    rpt = m_pad // NW
    gch = next(g for g in range(min(128, rpt), 0, -8) if rpt % g == 0)
    idx = _pad_idx(res_n_id, m_pad)
    xt, xtu, xtus = _gather3(xw, xwu, xwus, idx, m_pad, gch)

    xtu_s = jnp.concatenate([xtus[:, :wc], xtus[:, wc:]], axis=0)
    xsv_s = jnp.concatenate([xwv[:, :wc], xwv[:, wc:]], axis=0)

    src, tgt, wgt = _pad_edges(edge_index, edge_weight, n_tgt, 128)
    rpa = -(-(n_tgt + 1) // NS)
    zrows = jnp.zeros((rpa, wc + 16), F32)
    part = _edge_aggregate(xtu_s, xsv_s, src, tgt, wgt, zrows,
                           n_tgt, 128, m_pad, n_nodes)

    return _post(xt, xtu, part, n_tgt)


def kernel(X, edge_index_0, edge_index_1, edge_weight_0, edge_weight_1,
           res_n_id_0, res_n_id_1, Wn1, We1, u1, v1, Wn2, We2, u2, v2):
    b, n0, d_in = X.shape
    n1 = res_n_id_0.shape[0]
    n2 = res_n_id_1.shape[0]

    xf = jnp.transpose(X, (1, 0, 2)).reshape(n0, b * d_in)
    h1 = _layer(xf, n0, n1, edge_index_0, edge_weight_0, res_n_id_0,
                Wn1, We1, u1, v1, rb=1000)
    h2 = _layer(h1, n1, n2, edge_index_1, edge_weight_1, res_n_id_1,
                Wn2, We2, u2, v2, rb=1000)

    d_out = Wn2.shape[1]
    return jnp.transpose(h2.reshape(n2, b, d_out), (1, 0, 2))


# speedup vs baseline: 7.0992x; 7.0992x over previous
import functools

import jax
import jax.numpy as jnp
from jax import lax
from jax.experimental import pallas as pl
from jax.experimental.pallas import tpu as pltpu
from jax.experimental.pallas import tpu_sc as plsc

F32 = jnp.float32
I32 = jnp.int32

B = 2
NC = 2
NS = 16
NW = NC * NS


def _mm3(xf, Wn, u, v, wes, rb):
    n, k = xf.shape
    cin = k // B
    c = Wn.shape[1]
    wout = B * c

    def body(x_ref, wn_ref, u_ref, v_ref, ws_ref, o1, o2, o3, o4):
        for b in range(B):
            cs = slice(b * c, (b + 1) * c)
            xb = x_ref[:, b * cin:(b + 1) * cin]
            xw = jnp.dot(xb, wn_ref[...], preferred_element_type=F32)
            xwu = jnp.dot(xw, u_ref[...], preferred_element_type=F32)
            o1[:, cs] = xw
            o2[:, cs] = xwu
            o3[:, cs] = xwu * ws_ref[:, cs]
            o4[:, cs] = jnp.dot(xw, v_ref[...], preferred_element_type=F32)

    outs = [jax.ShapeDtypeStruct((n, wout), F32)] * 4
    return pl.pallas_call(
        body,
        grid=(n // rb,),
        in_specs=[
            pl.BlockSpec((rb, k), lambda i: (i, 0)),
            pl.BlockSpec((cin, c), lambda i: (0, 0)),
            pl.BlockSpec((c, c), lambda i: (0, 0)),
            pl.BlockSpec((c, c), lambda i: (0, 0)),
            pl.BlockSpec((1, wout), lambda i: (0, 0)),
        ],
        out_specs=[pl.BlockSpec((rb, wout), lambda i: (i, 0))] * 4,
        out_shape=outs,
    )(xf, Wn, u, v, wes)


def _gather3(tab1, tab2, tab3, idx, m_pad, ch):
    w = tab1.shape[1]
    rpt = m_pad // NW
    nch = rpt // ch
    mesh = plsc.VectorSubcoreMesh(core_axis_name="c", subcore_axis_name="s")

    @functools.partial(
        pl.kernel,
        mesh=mesh,
        compiler_params=pltpu.CompilerParams(use_tc_tiling_on_sc=False),
        out_type=[jax.ShapeDtypeStruct((m_pad, w), F32)] * 3,
        scratch_types=[
            pltpu.VMEM((ch,), I32),
            pltpu.VMEM((ch, w), F32),
            pltpu.VMEM((ch, w), F32),
            pltpu.VMEM((ch, w), F32),
            pltpu.SemaphoreType.DMA,
            pltpu.SemaphoreType.DMA,
            pltpu.SemaphoreType.DMA,
        ],
    )
    def k(t1, t2, t3, idx_hbm, o1, o2, o3, idx_v, b1, b2, b3, s1, s2, s3):
        cid = lax.axis_index("c")
        sid = lax.axis_index("s")
        base = (cid * NS + sid) * rpt

        def chunk(j, carry):
            off = base + j * ch
            pltpu.sync_copy(idx_hbm.at[pl.ds(off, ch)], idx_v)
            c1 = pltpu.async_copy(t1.at[idx_v], b1, s1)
            c2 = pltpu.async_copy(t2.at[idx_v], b2, s2)
            c3 = pltpu.async_copy(t3.at[idx_v], b3, s3)
            c1.wait()
            c2.wait()
            c3.wait()
            pltpu.sync_copy(b1, o1.at[pl.ds(off, ch)])
            pltpu.sync_copy(b2, o2.at[pl.ds(off, ch)])
            pltpu.sync_copy(b3, o3.at[pl.ds(off, ch)])
            return carry

        lax.fori_loop(0, nch, chunk, 0)

    return k(tab1, tab2, tab3, idx)


def _edge_aggregate(xtu_s, xsv_s, src, tgt, wgt, zrows,
                    nt, ch, n_xtu, n_xsv):
    wc = xtu_s.shape[1]
    wp = wc + 16
    e_pad = src.shape[0]
    tile_e = e_pad // NS
    nch = tile_e // ch
    rpa = zrows.shape[0]
    agr = NS * rpa
    nreg = wc // 16
    mesh = plsc.VectorSubcoreMesh(core_axis_name="c", subcore_axis_name="s")

    assert nch % 2 == 0 and nch >= 4

    @functools.partial(
        pl.kernel,
        mesh=mesh,
        compiler_params=pltpu.CompilerParams(use_tc_tiling_on_sc=False),
        out_type=jax.ShapeDtypeStruct((NC, agr, wp), F32),
        scratch_types=[
            pltpu.VMEM((2, ch), I32),
            pltpu.VMEM((2, ch), I32),
            pltpu.VMEM((2, ch), I32),
            pltpu.VMEM((2, ch), F32),
            pltpu.VMEM((2, ch, wc), F32),
            pltpu.VMEM((2, ch, wc), F32),
            pltpu.VMEM((2, ch, wp), F32),
            pltpu.VMEM_SHARED((agr, wp), F32),
            pltpu.SemaphoreType.DMA,
            pltpu.SemaphoreType.DMA,
            pltpu.SemaphoreType.DMA,
            pltpu.SemaphoreType.DMA,
            pltpu.SemaphoreType.DMA,
            pltpu.SemaphoreType.DMA,
        ],
    )
    def k(xtu, xsv, src_h, tgt_h, w_h, z_h, part,
          tgtn, tgto, srco, wv, bt, bs, msg, acc,
          st0, st1, ss0, ss1, sa0, sa1):
        cid = lax.axis_index("c")
        sid = lax.axis_index("s")
        base = sid * tile_e
        dt = cid * n_xtu
        doff = cid * n_xsv
        st = (st0, st1)
        ss = (ss0, ss1)
        sa = (sa0, sa1)

        pltpu.sync_copy(z_h, acc.at[pl.ds(sid * rpa, rpa)])
        plsc.subcore_barrier()

        one0 = jnp.where(lax.iota(I32, 16) == 0,
                         jnp.float32(1.0), jnp.float32(0.0))

        def init_cnt(e, carry):
            msg[0, e, pl.ds(wc, 16)] = one0
            msg[1, e, pl.ds(wc, 16)] = one0
            return carry

        lax.fori_loop(0, ch, init_cnt, 0)

        def fire(j, p):
            off = base + j * ch
            pltpu.sync_copy(tgt_h.at[pl.ds(off, ch)], tgto.at[p])
            pltpu.sync_copy(src_h.at[pl.ds(off, ch)], srco.at[p])
            pltpu.sync_copy(w_h.at[pl.ds(off, ch)], wv.at[p])
            for i in range(ch // 16):
                sl = pl.ds(16 * i, 16)
                tgto[p, sl] = tgto[p, sl] + dt
                srco[p, sl] = srco[p, sl] + doff
            pltpu.async_copy(xtu.at[tgto.at[p]], bt.at[p], st[p])
            pltpu.async_copy(xsv.at[srco.at[p]], bs.at[p], ss[p])

        def compute(j2, p):
            pltpu.make_async_copy(xtu.at[tgto.at[p]], bt.at[p], st[p]).wait()
            pltpu.make_async_copy(xsv.at[srco.at[p]], bs.at[p], ss[p]).wait()

            @pl.when(j2 >= 1)
            def _drain():
                pltpu.make_async_copy(
                    msg.at[p], acc.at[tgtn.at[p]], sa[p]).wait()

            for i in range(ch // 16):
                sl = pl.ds(16 * i, 16)
                tgtn[p, sl] = tgto[p, sl] - dt

            @plsc.parallel_loop(0, ch, unroll=4 if nreg == 1 else 2)
            def edge_body(e):
                w16 = wv[p, pl.ds((e // 16) * 16, 16)]
                we_s = jnp.take_along_axis(
                    w16, jnp.full((16,), e % 16, dtype=I32), axis=0)
                for r in range(nreg):
                    xu = bt[p, e, pl.ds(16 * r, 16)]
                    xv = bs[p, e, pl.ds(16 * r, 16)]
                    tn = xu * xv * we_s
                    msg[p, e, pl.ds(16 * r, 16)] = xv / (1.0 + jnp.exp(tn))

            pltpu.async_copy(msg.at[p], acc.at[tgtn.at[p]], sa[p], add=True)

        fire(0, 0)
        fire(1, 1)

        def pair(j2, carry):
            j = 2 * j2
            compute(j2, 0)

            @pl.when(j + 2 < nch)
            def _f0():
                fire(j + 2, 0)

            compute(j2, 1)

            @pl.when(j + 3 < nch)
            def _f1():
                fire(j + 3, 1)

            return carry

        lax.fori_loop(0, nch // 2, pair, 0)
        pltpu.make_async_copy(msg.at[0], acc.at[tgtn.at[0]], sa0).wait()
        pltpu.make_async_copy(msg.at[1], acc.at[tgtn.at[1]], sa1).wait()
        plsc.subcore_barrier()
        pltpu.sync_copy(acc.at[pl.ds(sid * rpa, rpa)],
                        part.at[cid, pl.ds(sid * rpa, rpa)])

    return k(xtu_s, xsv_s, src, tgt, wgt, zrows)


def _post(xt, xtu, part, nt):
    w = xt.shape[1]

    wc = w // 2

    def body(xt_ref, xtu_ref, p_ref, o_ref):
        sums = jnp.concatenate(
            [p_ref[0][:nt, :wc], p_ref[1][:nt, :wc]], axis=1)
        cnt = p_ref[0][:nt, wc:wc + 1]
        aggr = sums / jnp.maximum(cnt, 1.0)
        o = xtu_ref[:nt] + aggr
        mean = jnp.mean(o, axis=1, keepdims=True)
        var = jnp.mean((o - mean) ** 2, axis=1, keepdims=True)
        h = xt_ref[:nt] + (o - mean) / jnp.sqrt(var + 1e-5)
        o_ref[...] = jnp.where(h >= 0, h, 0.01 * h)

    return pl.pallas_call(
        body,
        out_shape=jax.ShapeDtypeStruct((nt, w), F32),
    )(xt, xtu, part)


def _pad_edges(edge_index, wgt, nt, ch):
    e = edge_index.shape[1]
    tile_e = -(-(e // NS) // (2 * ch)) * (2 * ch)
    pad = NS * tile_e - e
    src = jnp.concatenate([edge_index[0].astype(I32),
                           jnp.zeros((pad,), I32)])
    tgt = jnp.concatenate([edge_index[1].astype(I32),
                           jnp.full((pad,), nt, I32)])
    wp = jnp.concatenate([wgt, jnp.zeros((pad,), F32)])
    return src, tgt, wp


def _pad_idx(idx, m_pad):
    m = idx.shape[0]
    return jnp.concatenate([idx.astype(I32), jnp.zeros((m_pad - m,), I32)])


def _layer(xf, n_nodes, n_tgt, edge_index, edge_weight, res_n_id,
           Wn, We, u, v, rb):
    c = Wn.shape[1]
    w = B * c
    wc = w // 2

    wes = (-jnp.tile(We.reshape(-1), B)).reshape(1, w)
    xw, xwu, xwus, xwv = _mm3(xf, Wn, u, v, wes, rb)

    m_pad = -(-n_tgt // (NW * 16)) * (NW * 16)
    rpt = m_pad // NW
    gch = next(g for g in range(min(128, rpt), 0, -8) if rpt % g == 0)
    idx = _pad_idx(res_n_id, m_pad)
    xt, xtu, xtus = _gather3(xw, xwu, xwus, idx, m_pad, gch)

    xtu_s = jnp.concatenate([xtus[:, :wc], xtus[:, wc:]], axis=0)
    xsv_s = jnp.concatenate([xwv[:, :wc], xwv[:, wc:]], axis=0)

    src, tgt, wgt = _pad_edges(edge_index, edge_weight, n_tgt, 128)
    rpa = -(-(n_tgt + 1) // NS)
    zrows = jnp.zeros((rpa, wc + 16), F32)
    part = _edge_aggregate(xtu_s, xsv_s, src, tgt, wgt, zrows,
                           n_tgt, 128, m_pad, n_nodes)

    return _post(xt, xtu, part, n_tgt)


def kernel(X, edge_index_0, edge_index_1, edge_weight_0, edge_weight_1,
           res_n_id_0, res_n_id_1, Wn1, We1, u1, v1, Wn2, We2, u2, v2):
    b, n0, d_in = X.shape
    n1 = res_n_id_0.shape[0]
    n2 = res_n_id_1.shape[0]

    xf = jnp.transpose(X, (1, 0, 2)).reshape(n0, b * d_in)
    h1 = _layer(xf, n0, n1, edge_index_0, edge_weight_0, res_n_id_0,
                Wn1, We1, u1, v1, rb=1000)
    h2 = _layer(h1, n1, n2, edge_index_1, edge_weight_1, res_n_id_1,
                Wn2, We2, u2, v2, rb=1000)

    d_out = Wn2.shape[1]
    return jnp.transpose(h2.reshape(n2, b, d_out), (1, 0, 2))


# speedup vs baseline: 8.6667x; 1.2208x over previous
import functools

import jax
import jax.numpy as jnp
from jax import lax
from jax.experimental import pallas as pl
from jax.experimental.pallas import tpu as pltpu
from jax.experimental.pallas import tpu_sc as plsc

F32 = jnp.float32
I32 = jnp.int32

B = 2
NC = 2
NS = 16
NW = NC * NS


def _mm3(xf, Wn, u, v, wes, rb):
    n, k = xf.shape
    cin = k // B
    c = Wn.shape[1]
    wout = B * c

    def body(x_ref, wn_ref, u_ref, v_ref, ws_ref, o1, o2, o3, o4):
        for b in range(B):
            cs = slice(b * c, (b + 1) * c)
            xb = x_ref[:, b * cin:(b + 1) * cin]
            xw = jnp.dot(xb, wn_ref[...], preferred_element_type=F32)
            xwu = jnp.dot(xw, u_ref[...], preferred_element_type=F32)
            o1[:, cs] = xw
            o2[:, cs] = xwu
            o3[:, cs] = xwu * ws_ref[:, cs]
            o4[:, cs] = jnp.dot(xw, v_ref[...], preferred_element_type=F32)

    outs = [jax.ShapeDtypeStruct((n, wout), F32)] * 4
    return pl.pallas_call(
        body,
        grid=(n // rb,),
        in_specs=[
            pl.BlockSpec((rb, k), lambda i: (i, 0)),
            pl.BlockSpec((cin, c), lambda i: (0, 0)),
            pl.BlockSpec((c, c), lambda i: (0, 0)),
            pl.BlockSpec((c, c), lambda i: (0, 0)),
            pl.BlockSpec((1, wout), lambda i: (0, 0)),
        ],
        out_specs=[pl.BlockSpec((rb, wout), lambda i: (i, 0))] * 4,
        out_shape=outs,
    )(xf, Wn, u, v, wes)


def _gather3(tab1, tab2, tab3, idx, m_pad, ch):
    w = tab1.shape[1]
    rpt = m_pad // NW
    nch = rpt // ch
    mesh = plsc.VectorSubcoreMesh(core_axis_name="c", subcore_axis_name="s")

    @functools.partial(
        pl.kernel,
        mesh=mesh,
        compiler_params=pltpu.CompilerParams(use_tc_tiling_on_sc=False),
        out_type=[jax.ShapeDtypeStruct((m_pad, w), F32)] * 3,
        scratch_types=[
            pltpu.VMEM((ch,), I32),
            pltpu.VMEM((ch, w), F32),
            pltpu.VMEM((ch, w), F32),
            pltpu.VMEM((ch, w), F32),
            pltpu.SemaphoreType.DMA,
            pltpu.SemaphoreType.DMA,
            pltpu.SemaphoreType.DMA,
        ],
    )
    def k(t1, t2, t3, idx_hbm, o1, o2, o3, idx_v, b1, b2, b3, s1, s2, s3):
        cid = lax.axis_index("c")
        sid = lax.axis_index("s")
        base = (cid * NS + sid) * rpt

        def chunk(j, carry):
            off = base + j * ch
            pltpu.sync_copy(idx_hbm.at[pl.ds(off, ch)], idx_v)
            c1 = pltpu.async_copy(t1.at[idx_v], b1, s1)
            c2 = pltpu.async_copy(t2.at[idx_v], b2, s2)
            c3 = pltpu.async_copy(t3.at[idx_v], b3, s3)
            c1.wait()
            c2.wait()
            c3.wait()
            pltpu.sync_copy(b1, o1.at[pl.ds(off, ch)])
            pltpu.sync_copy(b2, o2.at[pl.ds(off, ch)])
            pltpu.sync_copy(b3, o3.at[pl.ds(off, ch)])
            return carry

        lax.fori_loop(0, nch, chunk, 0)

    return k(tab1, tab2, tab3, idx)


def _edge_aggregate(xtu_s, xsv_s, src, tgt, wgt, zrows,
                    nt, ch, n_xtu, n_xsv):
    wc = xtu_s.shape[1]
    wp = wc + 16
    e_pad = src.shape[0]
    tile_e = e_pad // NS
    nch = tile_e // ch
    rpa = zrows.shape[0]
    agr = NS * rpa
    nreg = wc // 16
    mesh = plsc.VectorSubcoreMesh(core_axis_name="c", subcore_axis_name="s")

    assert nch % 2 == 0 and nch >= 4

    @functools.partial(
        pl.kernel,
        mesh=mesh,
        compiler_params=pltpu.CompilerParams(use_tc_tiling_on_sc=False),
        out_type=jax.ShapeDtypeStruct((NC, agr, wp), F32),
        scratch_types=[
            pltpu.VMEM((2, ch), I32),
            pltpu.VMEM((2, ch), I32),
            pltpu.VMEM((2, ch), I32),
            pltpu.VMEM((2, ch), F32),
            pltpu.VMEM((2, ch, wc), F32),
            pltpu.VMEM((2, ch, wc), F32),
            pltpu.VMEM((2, ch, wp), F32),
            pltpu.VMEM_SHARED((agr, wp), F32),
            pltpu.SemaphoreType.DMA,
            pltpu.SemaphoreType.DMA,
            pltpu.SemaphoreType.DMA,
            pltpu.SemaphoreType.DMA,
            pltpu.SemaphoreType.DMA,
            pltpu.SemaphoreType.DMA,
            pltpu.SemaphoreType.DMA,
            pltpu.SemaphoreType.DMA,
            pltpu.SemaphoreType.DMA,
        ],
    )
    def k(xtu, xsv, src_h, tgt_h, w_h, z_h, part,
          tgtn, tgto, srco, wv, bt, bs, msg, acc,
          st0, st1, ss0, ss1, sa0, sa1, si0, si1, si2):
        cid = lax.axis_index("c")
        sid = lax.axis_index("s")
        base = sid * tile_e
        dt = cid * n_xtu
        doff = cid * n_xsv
        st = (st0, st1)
        ss = (ss0, ss1)
        sa = (sa0, sa1)

        pltpu.sync_copy(z_h, acc.at[pl.ds(sid * rpa, rpa)])
        plsc.subcore_barrier()

        one0 = jnp.where(lax.iota(I32, 16) == 0,
                         jnp.float32(1.0), jnp.float32(0.0))

        def init_cnt(e, carry):
            msg[0, e, pl.ds(wc, 16)] = one0
            msg[1, e, pl.ds(wc, 16)] = one0
            return carry

        lax.fori_loop(0, ch, init_cnt, 0)

        def fire(j, p):
            off = base + j * ch
            c1 = pltpu.async_copy(tgt_h.at[pl.ds(off, ch)], tgto.at[p], si0)
            c2 = pltpu.async_copy(src_h.at[pl.ds(off, ch)], srco.at[p], si1)
            c3 = pltpu.async_copy(w_h.at[pl.ds(off, ch)], wv.at[p], si2)
            c1.wait()
            c2.wait()
            c3.wait()
            for i in range(ch // 16):
                sl = pl.ds(16 * i, 16)
                tgto[p, sl] = tgto[p, sl] + dt
                srco[p, sl] = srco[p, sl] + doff
            pltpu.async_copy(xtu.at[tgto.at[p]], bt.at[p], st[p])
            pltpu.async_copy(xsv.at[srco.at[p]], bs.at[p], ss[p])

        def compute(j2, p):
            pltpu.make_async_copy(xtu.at[tgto.at[p]], bt.at[p], st[p]).wait()
            pltpu.make_async_copy(xsv.at[srco.at[p]], bs.at[p], ss[p]).wait()

            @pl.when(j2 >= 1)
            def _drain():
                pltpu.make_async_copy(
                    msg.at[p], acc.at[tgtn.at[p]], sa[p]).wait()

            for i in range(ch // 16):
                sl = pl.ds(16 * i, 16)
                tgtn[p, sl] = tgto[p, sl] - dt

            @plsc.parallel_loop(0, ch, unroll=8 if nreg == 1 else 4)
            def edge_body(e):
                w16 = wv[p, pl.ds((e // 16) * 16, 16)]
                we_s = jnp.take_along_axis(
                    w16, jnp.full((16,), e % 16, dtype=I32), axis=0)
                for r in range(nreg):
                    xu = bt[p, e, pl.ds(16 * r, 16)]
                    xv = bs[p, e, pl.ds(16 * r, 16)]
                    tn = xu * xv * we_s
                    msg[p, e, pl.ds(16 * r, 16)] = xv / (1.0 + jnp.exp(tn))

            pltpu.async_copy(msg.at[p], acc.at[tgtn.at[p]], sa[p], add=True)

        fire(0, 0)
        fire(1, 1)

        def pair(j2, carry):
            j = 2 * j2
            compute(j2, 0)

            @pl.when(j + 2 < nch)
            def _f0():
                fire(j + 2, 0)

            compute(j2, 1)

            @pl.when(j + 3 < nch)
            def _f1():
                fire(j + 3, 1)

            return carry

        lax.fori_loop(0, nch // 2, pair, 0)
        pltpu.make_async_copy(msg.at[0], acc.at[tgtn.at[0]], sa0).wait()
        pltpu.make_async_copy(msg.at[1], acc.at[tgtn.at[1]], sa1).wait()
        plsc.subcore_barrier()
        pltpu.sync_copy(acc.at[pl.ds(sid * rpa, rpa)],
                        part.at[cid, pl.ds(sid * rpa, rpa)])

    return k(xtu_s, xsv_s, src, tgt, wgt, zrows)


def _post(xt, xtu, part, nt):
    w = xt.shape[1]

    wc = w // 2

    def body(xt_ref, xtu_ref, p_ref, o_ref):
        sums = jnp.concatenate(
            [p_ref[0][:nt, :wc], p_ref[1][:nt, :wc]], axis=1)
        cnt = p_ref[0][:nt, wc:wc + 1]
        aggr = sums / jnp.maximum(cnt, 1.0)
        o = xtu_ref[:nt] + aggr
        mean = jnp.mean(o, axis=1, keepdims=True)
        var = jnp.mean((o - mean) ** 2, axis=1, keepdims=True)
        h = xt_ref[:nt] + (o - mean) / jnp.sqrt(var + 1e-5)
        o_ref[...] = jnp.where(h >= 0, h, 0.01 * h)

    return pl.pallas_call(
        body,
        out_shape=jax.ShapeDtypeStruct((nt, w), F32),
    )(xt, xtu, part)


def _pad_edges(edge_index, wgt, nt, ch):
    e = edge_index.shape[1]
    tile_e = -(-(e // NS) // (2 * ch)) * (2 * ch)
    pad = NS * tile_e - e
    src = jnp.concatenate([edge_index[0].astype(I32),
                           jnp.zeros((pad,), I32)])
    tgt = jnp.concatenate([edge_index[1].astype(I32),
                           jnp.full((pad,), nt, I32)])
    wp = jnp.concatenate([wgt, jnp.zeros((pad,), F32)])
    return src, tgt, wp


def _pad_idx(idx, m_pad):
    m = idx.shape[0]
    return jnp.concatenate([idx.astype(I32), jnp.zeros((m_pad - m,), I32)])


def _layer(xf, n_nodes, n_tgt, edge_index, edge_weight, res_n_id,
           Wn, We, u, v, rb):
    c = Wn.shape[1]
    w = B * c
    wc = w // 2

    wes = (-jnp.tile(We.reshape(-1), B)).reshape(1, w)
    xw, xwu, xwus, xwv = _mm3(xf, Wn, u, v, wes, rb)

    m_pad = -(-n_tgt // (NW * 16)) * (NW * 16)
    rpt = m_pad // NW
    gch = next(g for g in range(min(128, rpt), 0, -8) if rpt % g == 0)
    idx = _pad_idx(res_n_id, m_pad)
    xt, xtu, xtus = _gather3(xw, xwu, xwus, idx, m_pad, gch)

    xtu_s = jnp.concatenate([xtus[:, :wc], xtus[:, wc:]], axis=0)
    xsv_s = jnp.concatenate([xwv[:, :wc], xwv[:, wc:]], axis=0)

    src, tgt, wgt = _pad_edges(edge_index, edge_weight, n_tgt, 128)
    rpa = -(-(n_tgt + 1) // NS)
    zrows = jnp.zeros((rpa, wc + 16), F32)
    part = _edge_aggregate(xtu_s, xsv_s, src, tgt, wgt, zrows,
                           n_tgt, 128, m_pad, n_nodes)

    return _post(xt, xtu, part, n_tgt)


def kernel(X, edge_index_0, edge_index_1, edge_weight_0, edge_weight_1,
           res_n_id_0, res_n_id_1, Wn1, We1, u1, v1, Wn2, We2, u2, v2):
    b, n0, d_in = X.shape
    n1 = res_n_id_0.shape[0]
    n2 = res_n_id_1.shape[0]

    xf = jnp.transpose(X, (1, 0, 2)).reshape(n0, b * d_in)
    h1 = _layer(xf, n0, n1, edge_index_0, edge_weight_0, res_n_id_0,
                Wn1, We1, u1, v1, rb=1000)
    h2 = _layer(h1, n1, n2, edge_index_1, edge_weight_1, res_n_id_1,
                Wn2, We2, u2, v2, rb=1000)

    d_out = Wn2.shape[1]
    return jnp.transpose(h2.reshape(n2, b, d_out), (1, 0, 2))
